# baseline (speedup 1.0000x reference)
import functools
import math

import jax
import jax.numpy as jnp
from jax.experimental import pallas as pl
from jax.experimental.pallas import tpu as pltpu


def _gelu(x):
    theta_x = 1 + jnp.tanh(math.sqrt(2 / math.pi) * (x + 0.044715 * jnp.power(x, 3)))
    return 0.5 * x * theta_x


def _routing_kernel(x_ref, rw_ref, rb_ref, vals_ref, idx_ref, acc_ref, *, n_steps, n_total):
    step = pl.program_id(0)
    part = jnp.sum(x_ref[...], axis=0, keepdims=True)

    @pl.when(step == 0)
    def _init():
        acc_ref[...] = part

    @pl.when(step > 0)
    def _acc():
        acc_ref[...] += part

    @pl.when(step == n_steps - 1)
    def _finish():
        mean_x = acc_ref[...] / n_total
        scores = jnp.dot(mean_x, rw_ref[...],
                         preferred_element_type=jnp.float32) + rb_ref[...]
        m = jnp.max(scores, axis=1, keepdims=True)
        e = jnp.exp(scores - m)
        p = e / jnp.sum(e, axis=1, keepdims=True)
        i0 = jnp.argmax(p, axis=1)[0]
        v0 = jnp.max(p, axis=1)[0]
        col = jax.lax.broadcasted_iota(jnp.int32, p.shape, 1)
        p2 = jnp.where(col == i0, -jnp.inf, p)
        i1 = jnp.argmax(p2, axis=1)[0]
        v1 = jnp.max(p2, axis=1)[0]
        vals_ref[0] = v0
        vals_ref[1] = v1
        idx_ref[0] = i0.astype(jnp.int32)
        idx_ref[1] = i1.astype(jnp.int32)


def _ffn_kernel(idx_ref, xk_ref, fcw_ref, fcb_ref, pjw_ref, pjb_ref, vals_ref,
                out_ref, acc_ref, *, n_out, n_chunks, n_k):
    j = pl.program_id(0)
    c = pl.program_id(1)
    xv = xk_ref[0]
    h = jnp.dot(xv, fcw_ref[0], preferred_element_type=jnp.float32)
    h = _gelu(h + fcb_ref[0])
    y = jnp.dot(h, pjw_ref[0], preferred_element_type=jnp.float32)
    contrib = vals_ref[j] * y

    @pl.when(c == 0)
    def _bias():
        b = vals_ref[j] * pjb_ref[0]

        @pl.when(j == 0)
        def _init():
            acc_ref[...] = b

        @pl.when(j > 0)
        def _add():
            acc_ref[...] += b

    acc_ref[...] += contrib

    @pl.when((j == n_k - 1) & (c == n_chunks - 1))
    def _finish():
        row = acc_ref[...]
        out_ref[...] = jnp.broadcast_to(row, (n_out, row.shape[1]))


def kernel(x, fc_w, fc_b, proj_w, proj_b, route_w, route_b):
    B, N, D = x.shape
    E, _, F = fc_w.shape
    K = 2
    x2 = x[0]

    n_steps = 8
    tile = N // n_steps
    vals, idx = pl.pallas_call(
        functools.partial(_routing_kernel, n_steps=n_steps, n_total=float(N)),
        grid=(n_steps,),
        in_specs=[
            pl.BlockSpec((tile, D), lambda s: (s, 0)),
            pl.BlockSpec((D, E), lambda s: (0, 0)),
            pl.BlockSpec((1, E), lambda s: (0, 0)),
        ],
        out_specs=[
            pl.BlockSpec(memory_space=pltpu.SMEM),
            pl.BlockSpec(memory_space=pltpu.SMEM),
        ],
        out_shape=[
            jax.ShapeDtypeStruct((K,), jnp.float32),
            jax.ShapeDtypeStruct((K,), jnp.int32),
        ],
        scratch_shapes=[pltpu.VMEM((1, D), jnp.float32)],
    )(x2, route_w, route_b.reshape(1, E))

    n_chunks = 8
    fc = F // n_chunks
    out2 = pl.pallas_call(
        functools.partial(_ffn_kernel, n_out=N, n_chunks=n_chunks, n_k=K),
        grid_spec=pltpu.PrefetchScalarGridSpec(
            num_scalar_prefetch=1,
            grid=(K, n_chunks),
            in_specs=[
                pl.BlockSpec((1, 1, D), lambda j, c, idx_ref: (j, 0, 0)),
                pl.BlockSpec((1, D, fc), lambda j, c, idx_ref: (idx_ref[j], 0, c)),
                pl.BlockSpec((1, 1, fc), lambda j, c, idx_ref: (idx_ref[j], 0, c)),
                pl.BlockSpec((1, fc, D), lambda j, c, idx_ref: (idx_ref[j], c, 0)),
                pl.BlockSpec((1, 1, D), lambda j, c, idx_ref: (idx_ref[j], 0, 0)),
                pl.BlockSpec(memory_space=pltpu.SMEM),
            ],
            out_specs=pl.BlockSpec((N, D), lambda j, c, idx_ref: (0, 0)),
            scratch_shapes=[pltpu.VMEM((1, D), jnp.float32)],
        ),
        out_shape=jax.ShapeDtypeStruct((N, D), jnp.float32),
    )(idx, x2[:K].reshape(K, 1, D), fc_w, fc_b.reshape(E, 1, F),
      proj_w, proj_b.reshape(E, 1, D), vals)

    return out2[None]


# speedup vs baseline: 1.2454x; 1.2454x over previous
import functools
import math

import jax
import jax.numpy as jnp
from jax.experimental import pallas as pl
from jax.experimental.pallas import tpu as pltpu


def _gelu(x):
    theta_x = 1 + jnp.tanh(math.sqrt(2 / math.pi) * (x + 0.044715 * jnp.power(x, 3)))
    return 0.5 * x * theta_x


def _fused_kernel(x_ref, xk_ref, rw_ref, rb_ref, fcb_ref, pjb_ref,
                  fcw_hbm, pjw_hbm, out_ref,
                  acc_ref, w1_ref, w2_ref,
                  s0, s1, s2, s3,
                  *, n_steps, n_total, n_out):
    step = pl.program_id(0)
    part = jnp.sum(x_ref[...], axis=0, keepdims=True)

    @pl.when(step == 0)
    def _init():
        acc_ref[...] = part

    @pl.when(step > 0)
    def _acc():
        acc_ref[...] += part

    @pl.when(step == n_steps - 1)
    def _finish():
        mean_x = acc_ref[...] / n_total
        scores = jnp.dot(mean_x, rw_ref[...],
                         preferred_element_type=jnp.float32) + rb_ref[...]
        m = jnp.max(scores, axis=1, keepdims=True)
        e = jnp.exp(scores - m)
        p = e / jnp.sum(e, axis=1, keepdims=True)
        i0 = jnp.argmax(p, axis=1)[0]
        v0 = jnp.max(p, axis=1)[0]
        col = jax.lax.broadcasted_iota(jnp.int32, p.shape, 1)
        p2 = jnp.where(col == i0, -jnp.inf, p)
        i1 = jnp.argmax(p2, axis=1)[0]
        v1 = jnp.max(p2, axis=1)[0]

        c0 = pltpu.make_async_copy(fcw_hbm.at[i0], w1_ref.at[0], s0)
        c1 = pltpu.make_async_copy(fcw_hbm.at[i1], w1_ref.at[1], s1)
        c2 = pltpu.make_async_copy(pjw_hbm.at[i0], w2_ref.at[0], s2)
        c3 = pltpu.make_async_copy(pjw_hbm.at[i1], w2_ref.at[1], s3)
        c0.start()
        c1.start()
        c2.start()
        c3.start()

        xv0 = xk_ref[0]
        xv1 = xk_ref[1]
        b1_0 = fcb_ref[i0]
        b1_1 = fcb_ref[i1]
        b2_0 = pjb_ref[i0]
        b2_1 = pjb_ref[i1]

        c0.wait()
        h0 = _gelu(jnp.dot(xv0, w1_ref[0],
                           preferred_element_type=jnp.float32) + b1_0)
        c1.wait()
        h1 = _gelu(jnp.dot(xv1, w1_ref[1],
                           preferred_element_type=jnp.float32) + b1_1)
        c2.wait()
        y0 = jnp.dot(h0, w2_ref[0], preferred_element_type=jnp.float32) + b2_0
        c3.wait()
        y1 = jnp.dot(h1, w2_ref[1], preferred_element_type=jnp.float32) + b2_1
        row = v0 * y0 + v1 * y1
        out_ref[...] = jnp.broadcast_to(row, (n_out, row.shape[1]))


def kernel(x, fc_w, fc_b, proj_w, proj_b, route_w, route_b):
    B, N, D = x.shape
    E, _, F = fc_w.shape
    K = 2
    x2 = x[0]

    n_steps = 8
    tile = N // n_steps
    out2 = pl.pallas_call(
        functools.partial(_fused_kernel, n_steps=n_steps, n_total=float(N),
                          n_out=N),
        grid=(n_steps,),
        in_specs=[
            pl.BlockSpec((tile, D), lambda s: (s, 0)),
            pl.BlockSpec((K, 1, D), lambda s: (0, 0, 0)),
            pl.BlockSpec((D, E), lambda s: (0, 0)),
            pl.BlockSpec((1, E), lambda s: (0, 0)),
            pl.BlockSpec((E, 1, F), lambda s: (0, 0, 0)),
            pl.BlockSpec((E, 1, D), lambda s: (0, 0, 0)),
            pl.BlockSpec(memory_space=pltpu.HBM),
            pl.BlockSpec(memory_space=pltpu.HBM),
        ],
        out_specs=pl.BlockSpec((N, D), lambda s: (0, 0)),
        out_shape=jax.ShapeDtypeStruct((N, D), jnp.float32),
        scratch_shapes=[
            pltpu.VMEM((1, D), jnp.float32),
            pltpu.VMEM((K, D, F), jnp.float32),
            pltpu.VMEM((K, F, D), jnp.float32),
            pltpu.SemaphoreType.DMA,
            pltpu.SemaphoreType.DMA,
            pltpu.SemaphoreType.DMA,
            pltpu.SemaphoreType.DMA,
        ],
    )(x2, x2[:K].reshape(K, 1, D), route_w, route_b.reshape(1, E),
      fc_b.reshape(E, 1, F), proj_b.reshape(E, 1, D), fc_w, proj_w)

    return out2[None]
